# Initial kernel scaffold; baseline (speedup 1.0000x reference)
import functools
import math

import jax
import jax.numpy as jnp
from jax import lax
from jax.experimental import pallas as pl
from jax.experimental.pallas import tpu as pltpu

N = 10000
NPAD = 10240
E = 320000
NG = 16
LAYER_DIMS = [(128, 512), (512, 256), (256, 64), (64, 32)]

_HI = lax.Precision.HIGHEST


def _mm_kernel(x_ref, w_ref, b_ref, o_ref):
    o_ref[...] = (
        lax.dot_general(x_ref[...], w_ref[...], (((1,), (0,)), ((), ())),
                        precision=_HI)
        + b_ref[...]
    )


def _qkvs_matmul(h, W, b):
    din, dout4 = W.shape
    blk = 256
    return pl.pallas_call(
        _mm_kernel,
        grid=(NPAD // blk,),
        in_specs=[
            pl.BlockSpec((blk, din), lambda i: (i, 0)),
            pl.BlockSpec((din, dout4), lambda i: (0, 0)),
            pl.BlockSpec((1, dout4), lambda i: (0, 0)),
        ],
        out_specs=pl.BlockSpec((blk, dout4), lambda i: (i, 0)),
        out_shape=jax.ShapeDtypeStruct((NPAD, dout4), jnp.float32),
    )(h, W, b)


def _elu(v):
    return jnp.where(v > 0, v, jnp.expm1(v))


def _pool_kernel(h_ref, batch_ref, wg_ref, bg_ref, wf_ref, bf_ref, o_ref):
    h = h_ref[...]
    gate = (
        lax.dot_general(h, wg_ref[...], (((1,), (0,)), ((), ())), precision=_HI)
        + bg_ref[...]
    )
    batch = batch_ref[...]
    onehot = (batch.reshape(1, NPAD) ==
              lax.broadcasted_iota(jnp.int32, (NG, NPAD), 0))
    gate_row = gate.reshape(1, NPAD)
    neg = jnp.float32(-jnp.inf)
    gm = jnp.max(jnp.where(onehot, gate_row, neg), axis=1, keepdims=True)
    gm = jnp.where(jnp.isfinite(gm), gm, 0.0)
    ge = jnp.where(onehot, jnp.exp(gate_row - gm), 0.0)
    gs = jnp.sum(ge, axis=1, keepdims=True)
    wnode = ge / (gs + 1e-16)
    pooled = lax.dot_general(wnode, h, (((1,), (0,)), ((), ())), precision=_HI)
    o_ref[...] = (
        lax.dot_general(pooled, wf_ref[...], (((1,), (0,)), ((), ())),
                        precision=_HI)
        + bf_ref[...]
    )


def _pooling(h4, batch_pad, Wg, bg, Wf, bf):
    return pl.pallas_call(
        _pool_kernel,
        in_specs=[
            pl.BlockSpec((NPAD, 32), lambda: (0, 0)),
            pl.BlockSpec((NPAD, 1), lambda: (0, 0)),
            pl.BlockSpec((32, 1), lambda: (0, 0)),
            pl.BlockSpec((1, 1), lambda: (0, 0)),
            pl.BlockSpec((32, 3), lambda: (0, 0)),
            pl.BlockSpec((1, 3), lambda: (0, 0)),
        ],
        out_specs=pl.BlockSpec((NG, 3), lambda: (0, 0)),
        out_shape=jax.ShapeDtypeStruct((NG, 3), jnp.float32),
    )(h4, batch_pad.reshape(NPAD, 1), Wg, bg.reshape(1, 1), Wf,
      bf.reshape(1, 3))


def kernel(x, edge_index, batch, params):
    src = edge_index[0]
    dst = edge_index[1]
    h = jnp.pad(x, ((0, NPAD - N), (0, 0)))
    batch_pad = jnp.pad(batch, (0, NPAD - N), constant_values=NG)

    for p, (din, dout) in zip(params["convs"], LAYER_DIMS):
        Wcat = jnp.concatenate([p["Wq"], p["Wk"], p["Wv"], p["Ws"]], axis=1)
        bcat = jnp.concatenate([p["bq"], p["bk"], p["bv"], p["bs"]])
        qkvs = _qkvs_matmul(h, Wcat, bcat.reshape(1, 4 * dout))
        q = qkvs[:N, 0 * dout:1 * dout]
        k = qkvs[:N, 1 * dout:2 * dout]
        v = qkvs[:N, 2 * dout:3 * dout]
        skip = qkvs[:, 3 * dout:4 * dout]

        logits = jnp.sum(q[dst] * k[src], axis=-1) / math.sqrt(float(dout))
        m = jax.ops.segment_max(logits, dst, num_segments=N)
        m = jnp.where(jnp.isfinite(m), m, 0.0)
        e = jnp.exp(logits - m[dst])
        s = jax.ops.segment_sum(e, dst, num_segments=N)
        alpha = e / (s[dst] + 1e-16)
        agg = jax.ops.segment_sum(v[src] * alpha[:, None], dst,
                                  num_segments=N)
        agg = jnp.pad(agg, ((0, NPAD - N), (0, 0)))
        h = _elu(agg + skip)

    return _pooling(h, batch_pad, params["Wg"], params["bg"], params["Wf"],
                    params["bf"])


# speedup vs baseline: 1.0943x; 1.0943x over previous
import functools
import math

import jax
import jax.numpy as jnp
from jax import lax
from jax.experimental import pallas as pl
from jax.experimental.pallas import tpu as pltpu

N = 10000
NPAD = 10240
E = 320000
NG = 16
LAYER_DIMS = [(128, 512), (512, 256), (256, 64), (64, 32)]

_HI = lax.Precision.HIGHEST


def _mm_kernel(x_ref, w_ref, b_ref, o_ref):
    o_ref[...] = (
        lax.dot_general(x_ref[...], w_ref[...], (((1,), (0,)), ((), ())),
                        precision=_HI)
        + b_ref[...]
    )


def _qkvs_matmul(h, W, b):
    din, dout4 = W.shape
    blk = 256
    return pl.pallas_call(
        _mm_kernel,
        grid=(NPAD // blk,),
        in_specs=[
            pl.BlockSpec((blk, din), lambda i: (i, 0)),
            pl.BlockSpec((din, dout4), lambda i: (0, 0)),
            pl.BlockSpec((1, dout4), lambda i: (0, 0)),
        ],
        out_specs=pl.BlockSpec((blk, dout4), lambda i: (i, 0)),
        out_shape=jax.ShapeDtypeStruct((NPAD, dout4), jnp.float32),
    )(h, W, b)


def _elu(v):
    return jnp.where(v > 0, v, jnp.expm1(v))


def _pool_kernel(h_ref, batch_ref, wgt_ref, bg_ref, wf_ref, bf_ref, o_ref):
    h = h_ref[...]
    gate_row = (
        lax.dot_general(wgt_ref[...], h, (((1,), (1,)), ((), ())),
                        precision=_HI)
        + bg_ref[...]
    )
    batch_row = batch_ref[0:1, :]
    onehot = (batch_row ==
              lax.broadcasted_iota(jnp.int32, (NG, NPAD), 0))
    neg = jnp.float32(-jnp.inf)
    gm = jnp.max(jnp.where(onehot, gate_row, neg), axis=1, keepdims=True)
    gm = jnp.where(jnp.isfinite(gm), gm, 0.0)
    ge = jnp.where(onehot, jnp.exp(gate_row - gm), 0.0)
    gs = jnp.sum(ge, axis=1, keepdims=True)
    wnode = ge / (gs + 1e-16)
    pooled = lax.dot_general(wnode, h, (((1,), (0,)), ((), ())), precision=_HI)
    o_ref[...] = (
        lax.dot_general(pooled, wf_ref[...], (((1,), (0,)), ((), ())),
                        precision=_HI)
        + bf_ref[...]
    )


def _pooling(h4, batch_pad, Wg, bg, Wf, bf):
    batch8 = jnp.broadcast_to(batch_pad.reshape(1, NPAD), (8, NPAD))
    return pl.pallas_call(
        _pool_kernel,
        in_specs=[
            pl.BlockSpec((NPAD, 32), lambda: (0, 0)),
            pl.BlockSpec((8, NPAD), lambda: (0, 0)),
            pl.BlockSpec((1, 32), lambda: (0, 0)),
            pl.BlockSpec((1, 1), lambda: (0, 0)),
            pl.BlockSpec((32, 3), lambda: (0, 0)),
            pl.BlockSpec((1, 3), lambda: (0, 0)),
        ],
        out_specs=pl.BlockSpec((NG, 3), lambda: (0, 0)),
        out_shape=jax.ShapeDtypeStruct((NG, 3), jnp.float32),
    )(h4, batch8, Wg.reshape(1, 32), bg.reshape(1, 1), Wf,
      bf.reshape(1, 3))


def kernel(x, edge_index, batch, params):
    src = edge_index[0]
    dst = edge_index[1]
    h = jnp.pad(x, ((0, NPAD - N), (0, 0)))
    batch_pad = jnp.pad(batch, (0, NPAD - N), constant_values=NG)

    for p, (din, dout) in zip(params["convs"], LAYER_DIMS):
        Wcat = jnp.concatenate([p["Wq"], p["Wk"], p["Wv"], p["Ws"]], axis=1)
        bcat = jnp.concatenate([p["bq"], p["bk"], p["bv"], p["bs"]])
        qkvs = _qkvs_matmul(h, Wcat, bcat.reshape(1, 4 * dout))
        q = qkvs[:N, 0 * dout:1 * dout]
        k = qkvs[:N, 1 * dout:2 * dout]
        v = qkvs[:N, 2 * dout:3 * dout]
        skip = qkvs[:, 3 * dout:4 * dout]

        logits = jnp.sum(q[dst] * k[src], axis=-1) / math.sqrt(float(dout))
        m = jax.ops.segment_max(logits, dst, num_segments=N)
        m = jnp.where(jnp.isfinite(m), m, 0.0)
        e = jnp.exp(logits - m[dst])
        s = jax.ops.segment_sum(e, dst, num_segments=N)
        alpha = e / (s[dst] + 1e-16)
        agg = jax.ops.segment_sum(v[src] * alpha[:, None], dst,
                                  num_segments=N)
        agg = jnp.pad(agg, ((0, NPAD - N), (0, 0)))
        h = _elu(agg + skip)

    return _pooling(h, batch_pad, params["Wg"], params["bg"], params["Wf"],
                    params["bf"])
